# baseline (speedup 1.0000x reference)
import jax
import jax.numpy as jnp
from jax import lax
from jax.experimental import pallas as pl
from jax.experimental.pallas import tpu as pltpu
from jax.experimental.pallas import tpu_sc as plsc

N_NODES = 10000
N_EDGES = 160000
D_FEAT = 256
D_HID = 512
N_GRAPHS = 64

NC = 2
NT = 16
LCHUNK = 128
NCHUNK = D_HID // LCHUNK
EB = 128
EROWS = N_EDGES // EB
TBF = 80
TBL = EROWS - 15 * TBF

ROWS_PER_TILE = 632
SP_ROWS = NT * ROWS_PER_TILE

NB = 2
BN = N_NODES // NB


def _encoder_body(x_ref, w_ref, b_ref, o_ref):
    y = jnp.dot(x_ref[...], w_ref[...], preferred_element_type=jnp.float32)
    y = jnp.maximum(y + b_ref[...], 0.0)
    for c in range(NCHUNK):
        o_ref[c] = y[:, c * LCHUNK:(c + 1) * LCHUNK]


def _encode(x, W_enc, b_enc):
    return pl.pallas_call(
        _encoder_body,
        grid=(NB,),
        in_specs=[
            pl.BlockSpec((BN, D_FEAT), lambda i: (i, 0)),
            pl.BlockSpec((D_FEAT, D_HID), lambda i: (0, 0)),
            pl.BlockSpec((1, D_HID), lambda i: (0, 0)),
        ],
        out_specs=pl.BlockSpec((NCHUNK, BN, LCHUNK), lambda i: (0, i, 0)),
        out_shape=jax.ShapeDtypeStruct((NCHUNK, N_NODES, LCHUNK), jnp.float32),
    )(x, W_enc, b_enc.reshape(1, D_HID))


HB = TBF // 2


def _agg_body(h_ref, ei_ref, out_ref,
              spmem, zbuf, gbufa, gbufb, srcv, dstv, sema, semb):
    core = lax.axis_index("c")
    sub = lax.axis_index("s")

    def _z(i, _):
        for j in range(8):
            zbuf[i, pl.ds(j * 16, 16)] = jnp.zeros((16,), jnp.float32)
        return 0
    lax.fori_loop(0, 32, _z, 0)

    def _zero_own_rows():
        for k in range(19):
            pltpu.sync_copy(zbuf, spmem.at[pl.ds(sub * ROWS_PER_TILE + k * 32, 32)])
        pltpu.sync_copy(zbuf.at[pl.ds(0, 24)],
                        spmem.at[pl.ds(sub * ROWS_PER_TILE + 608, 24)])

    def _stage_half(half, sync):
        row0 = sub * TBF + half * HB
        if half == 0:
            pltpu.async_copy(ei_ref.at[0].at[pl.ds(row0, HB)], srcv, sema)
            pltpu.async_copy(ei_ref.at[1].at[pl.ds(row0, HB)], dstv, semb)
        else:
            @pl.when(sub < NT - 1)
            def _stage_full():
                pltpu.async_copy(ei_ref.at[0].at[pl.ds(row0, HB)], srcv, sema)
                pltpu.async_copy(ei_ref.at[1].at[pl.ds(row0, HB)], dstv, semb)

            last0 = (NT - 1) * TBF + HB

            @pl.when(sub == NT - 1)
            def _stage_last():
                pltpu.async_copy(ei_ref.at[0].at[pl.ds(last0, TBL - HB)],
                                 srcv.at[pl.ds(0, TBL - HB)], sema)
                pltpu.async_copy(ei_ref.at[1].at[pl.ds(last0, TBL - HB)],
                                 dstv.at[pl.ds(0, TBL - HB)], semb)
        if sync:
            _stage_wait(half)

    def _stage_wait(half):
        if half == 0:
            pltpu.make_async_copy(ei_ref.at[0].at[pl.ds(0, HB)], srcv, sema).wait()
            pltpu.make_async_copy(ei_ref.at[1].at[pl.ds(0, HB)], dstv, semb).wait()
        else:
            @pl.when(sub < NT - 1)
            def _wait_full():
                pltpu.make_async_copy(ei_ref.at[0].at[pl.ds(0, HB)], srcv,
                                      sema).wait()
                pltpu.make_async_copy(ei_ref.at[1].at[pl.ds(0, HB)], dstv,
                                      semb).wait()

            last0 = (NT - 1) * TBF + HB

            @pl.when(sub == NT - 1)
            def _wait_last():
                pltpu.make_async_copy(ei_ref.at[0].at[pl.ds(last0, TBL - HB)],
                                      srcv.at[pl.ds(0, TBL - HB)], sema).wait()
                pltpu.make_async_copy(ei_ref.at[1].at[pl.ds(last0, TBL - HB)],
                                      dstv.at[pl.ds(0, TBL - HB)], semb).wait()

    _stage_half(0, sync=False)
    _zero_own_rows()
    _stage_wait(0)
    plsc.subcore_barrier()

    for cc in range(2):
        c = core * 2 + cc

        halves = (0, 1) if cc == 0 else (1, 0)
        for hi, half in enumerate(halves):
            if hi == 1:
                _stage_half(half, sync=True)
            if half == 0:
                nb = HB
            else:
                nb = jnp.where(sub == NT - 1, TBL - HB, HB)

            def _gather(j, buf, sem):
                return pltpu.async_copy(h_ref.at[c].at[srcv.at[j]], buf, sem)

            def _gwait(buf, sem):
                pltpu.make_async_copy(h_ref.at[c].at[srcv.at[0]], buf, sem).wait()

            _gather(0, gbufa, sema)

            def _pair(k, _):
                j0 = 2 * k
                j1 = j0 + 1
                _gather(j1, gbufb, semb)
                _gwait(gbufa, sema)
                pltpu.sync_copy(gbufa, spmem.at[dstv.at[j0]], add=True)

                @pl.when(j1 + 1 < nb)
                def _next():
                    _gather(j1 + 1, gbufa, sema)

                _gwait(gbufb, semb)
                pltpu.sync_copy(gbufb, spmem.at[dstv.at[j1]], add=True)
                return 0

            lax.fori_loop(0, nb // 2, _pair, 0)
        plsc.subcore_barrier()

        base = sub * ROWS_PER_TILE
        pltpu.sync_copy(spmem.at[pl.ds(base, ROWS_PER_TILE)],
                        out_ref.at[c].at[pl.ds(base, ROWS_PER_TILE)])
        if cc == 0:
            plsc.subcore_barrier()


def _aggregate(h_chunks, ei3):
    k = pl.kernel(
        _agg_body,
        out_type=jax.ShapeDtypeStruct((NCHUNK, SP_ROWS, LCHUNK), jnp.float32),
        mesh=plsc.VectorSubcoreMesh(core_axis_name="c", subcore_axis_name="s"),
        scratch_types=[
            pltpu.VMEM_SHARED((SP_ROWS, LCHUNK), jnp.float32),
            pltpu.VMEM((32, 128), jnp.float32),
            pltpu.VMEM((EB, LCHUNK), jnp.float32),
            pltpu.VMEM((EB, LCHUNK), jnp.float32),
            pltpu.VMEM((HB, EB), jnp.int32),
            pltpu.VMEM((HB, EB), jnp.int32),
            pltpu.SemaphoreType.DMA,
            pltpu.SemaphoreType.DMA,
        ],
    )
    return k(h_chunks, ei3)


def _self_body(h_ref, ws_ref, b_ref, o_ref):
    acc = jnp.zeros((BN, D_HID), jnp.float32) + b_ref[...]
    for c in range(NCHUNK):
        acc += jnp.dot(h_ref[c], ws_ref[c], preferred_element_type=jnp.float32)
    o_ref[...] = acc


def _self_term(h_chunks, W_self, b_gnn):
    return pl.pallas_call(
        _self_body,
        grid=(NB,),
        in_specs=[
            pl.BlockSpec((NCHUNK, BN, LCHUNK), lambda i: (0, i, 0)),
            pl.BlockSpec((NCHUNK, LCHUNK, D_HID), lambda i: (0, 0, 0)),
            pl.BlockSpec((1, D_HID), lambda i: (0, 0)),
        ],
        out_specs=pl.BlockSpec((BN, D_HID), lambda i: (i, 0)),
        out_shape=jax.ShapeDtypeStruct((N_NODES, D_HID), jnp.float32),
    )(h_chunks, W_self.reshape(NCHUNK, LCHUNK, D_HID), b_gnn.reshape(1, D_HID))


def _gnn_pool_body(agg_ref, s_ref, wm_ref, batch_ref, out_ref, cnt_ref):
    i = pl.program_id(0)
    acc = s_ref[...]
    for c in range(NCHUNK):
        a = agg_ref[c] if c % 2 == 0 else agg_ref[c] - agg_ref[c - 1]
        acc += jnp.dot(a, wm_ref[c], preferred_element_type=jnp.float32)
    nf = jnp.maximum(acc, 0.0)

    bvec = batch_ref[0, 0, :]
    gids = lax.broadcasted_iota(jnp.int32, (N_GRAPHS, BN), 0)
    maskT = (gids == bvec[None, :]).astype(jnp.float32)
    contrib = jnp.dot(maskT, nf, preferred_element_type=jnp.float32)
    cnt = jnp.sum(maskT, axis=1)

    @pl.when(i == 0)
    def _init():
        out_ref[...] = jnp.zeros_like(out_ref)
        cnt_ref[...] = jnp.zeros_like(cnt_ref)

    out_ref[...] += contrib
    cnt_ref[...] += jnp.broadcast_to(cnt[:, None], (N_GRAPHS, 128))

    @pl.when(i == NB - 1)
    def _fin():
        out_ref[...] = out_ref[...] / jnp.maximum(cnt_ref[:, 0:1], 1.0)


def _gnn_pool(agg_chunks, s, W_msg, batch3):
    return pl.pallas_call(
        _gnn_pool_body,
        grid=(NB,),
        in_specs=[
            pl.BlockSpec((NCHUNK, BN, LCHUNK), lambda i: (0, i, 0)),
            pl.BlockSpec((BN, D_HID), lambda i: (i, 0)),
            pl.BlockSpec((NCHUNK, LCHUNK, D_HID), lambda i: (0, 0, 0)),
            pl.BlockSpec((1, 1, BN), lambda i: (i, 0, 0)),
        ],
        out_specs=pl.BlockSpec((N_GRAPHS, D_HID), lambda i: (0, 0)),
        out_shape=jax.ShapeDtypeStruct((N_GRAPHS, D_HID), jnp.float32),
        scratch_shapes=[pltpu.VMEM((N_GRAPHS, 128), jnp.float32)],
    )(agg_chunks, s, W_msg.reshape(NCHUNK, LCHUNK, D_HID), batch3)


@jax.jit
def kernel(x, edge_index, batch, W_enc, b_enc, W_msg, W_self, b_gnn):
    ei3 = edge_index.astype(jnp.int32).reshape(2, EROWS, EB)
    batch3 = batch.astype(jnp.int32).reshape(NB, 1, BN)

    h_chunks = _encode(x, W_enc, b_enc)
    agg_chunks = _aggregate(h_chunks, ei3)
    s = _self_term(h_chunks, W_self, b_gnn)
    return _gnn_pool(agg_chunks, s, W_msg, batch3)


# speedup vs baseline: 1.0014x; 1.0014x over previous
import jax
import jax.numpy as jnp
from jax import lax
from jax.experimental import pallas as pl
from jax.experimental.pallas import tpu as pltpu
from jax.experimental.pallas import tpu_sc as plsc

N_NODES = 10000
N_EDGES = 160000
D_FEAT = 256
D_HID = 512
N_GRAPHS = 64

NC = 2
NT = 16
LCHUNK = 128
NCHUNK = D_HID // LCHUNK
EB = 128
EROWS = N_EDGES // EB
TBF = 80
TBL = EROWS - 15 * TBF

ROWS_PER_TILE = 632
SP_ROWS = NT * ROWS_PER_TILE

NB = 5
BN = N_NODES // NB


def _encoder_body(x_ref, w_ref, b_ref, o_ref):
    y = jnp.dot(x_ref[...], w_ref[...], preferred_element_type=jnp.float32)
    y = jnp.maximum(y + b_ref[...], 0.0)
    for c in range(NCHUNK):
        o_ref[c] = y[:, c * LCHUNK:(c + 1) * LCHUNK]


def _encode(x, W_enc, b_enc):
    return pl.pallas_call(
        _encoder_body,
        grid=(NB,),
        in_specs=[
            pl.BlockSpec((BN, D_FEAT), lambda i: (i, 0)),
            pl.BlockSpec((D_FEAT, D_HID), lambda i: (0, 0)),
            pl.BlockSpec((1, D_HID), lambda i: (0, 0)),
        ],
        out_specs=pl.BlockSpec((NCHUNK, BN, LCHUNK), lambda i: (0, i, 0)),
        out_shape=jax.ShapeDtypeStruct((NCHUNK, N_NODES, LCHUNK), jnp.float32),
    )(x, W_enc, b_enc.reshape(1, D_HID))


HB = TBF // 2


def _agg_body(h_ref, ei_ref, out_ref,
              spmem, zbuf, gbufa, gbufb, srcv, dstv, sema, semb):
    core = lax.axis_index("c")
    sub = lax.axis_index("s")

    def _z(i, _):
        for j in range(8):
            zbuf[i, pl.ds(j * 16, 16)] = jnp.zeros((16,), jnp.float32)
        return 0
    lax.fori_loop(0, 32, _z, 0)

    def _zero_own_rows():
        for k in range(19):
            pltpu.sync_copy(zbuf, spmem.at[pl.ds(sub * ROWS_PER_TILE + k * 32, 32)])
        pltpu.sync_copy(zbuf.at[pl.ds(0, 24)],
                        spmem.at[pl.ds(sub * ROWS_PER_TILE + 608, 24)])

    def _stage_half(half, sync):
        row0 = sub * TBF + half * HB
        if half == 0:
            pltpu.async_copy(ei_ref.at[0].at[pl.ds(row0, HB)], srcv, sema)
            pltpu.async_copy(ei_ref.at[1].at[pl.ds(row0, HB)], dstv, semb)
        else:
            @pl.when(sub < NT - 1)
            def _stage_full():
                pltpu.async_copy(ei_ref.at[0].at[pl.ds(row0, HB)], srcv, sema)
                pltpu.async_copy(ei_ref.at[1].at[pl.ds(row0, HB)], dstv, semb)

            last0 = (NT - 1) * TBF + HB

            @pl.when(sub == NT - 1)
            def _stage_last():
                pltpu.async_copy(ei_ref.at[0].at[pl.ds(last0, TBL - HB)],
                                 srcv.at[pl.ds(0, TBL - HB)], sema)
                pltpu.async_copy(ei_ref.at[1].at[pl.ds(last0, TBL - HB)],
                                 dstv.at[pl.ds(0, TBL - HB)], semb)
        if sync:
            _stage_wait(half)

    def _stage_wait(half):
        if half == 0:
            pltpu.make_async_copy(ei_ref.at[0].at[pl.ds(0, HB)], srcv, sema).wait()
            pltpu.make_async_copy(ei_ref.at[1].at[pl.ds(0, HB)], dstv, semb).wait()
        else:
            @pl.when(sub < NT - 1)
            def _wait_full():
                pltpu.make_async_copy(ei_ref.at[0].at[pl.ds(0, HB)], srcv,
                                      sema).wait()
                pltpu.make_async_copy(ei_ref.at[1].at[pl.ds(0, HB)], dstv,
                                      semb).wait()

            last0 = (NT - 1) * TBF + HB

            @pl.when(sub == NT - 1)
            def _wait_last():
                pltpu.make_async_copy(ei_ref.at[0].at[pl.ds(last0, TBL - HB)],
                                      srcv.at[pl.ds(0, TBL - HB)], sema).wait()
                pltpu.make_async_copy(ei_ref.at[1].at[pl.ds(last0, TBL - HB)],
                                      dstv.at[pl.ds(0, TBL - HB)], semb).wait()

    _stage_half(0, sync=False)
    _zero_own_rows()
    _stage_wait(0)
    plsc.subcore_barrier()

    for cc in range(2):
        c = core * 2 + cc

        halves = (0, 1) if cc == 0 else (1, 0)
        for hi, half in enumerate(halves):
            if hi == 1:
                _stage_half(half, sync=True)
            if half == 0:
                nb = HB
            else:
                nb = jnp.where(sub == NT - 1, TBL - HB, HB)

            def _gather(j, buf, sem):
                return pltpu.async_copy(h_ref.at[c].at[srcv.at[j]], buf, sem)

            def _gwait(buf, sem):
                pltpu.make_async_copy(h_ref.at[c].at[srcv.at[0]], buf, sem).wait()

            _gather(0, gbufa, sema)

            def _pair(k, _):
                j0 = 2 * k
                j1 = j0 + 1
                _gather(j1, gbufb, semb)
                _gwait(gbufa, sema)
                pltpu.sync_copy(gbufa, spmem.at[dstv.at[j0]], add=True)

                @pl.when(j1 + 1 < nb)
                def _next():
                    _gather(j1 + 1, gbufa, sema)

                _gwait(gbufb, semb)
                pltpu.sync_copy(gbufb, spmem.at[dstv.at[j1]], add=True)
                return 0

            lax.fori_loop(0, nb // 2, _pair, 0)
        plsc.subcore_barrier()

        base = sub * ROWS_PER_TILE
        pltpu.sync_copy(spmem.at[pl.ds(base, ROWS_PER_TILE)],
                        out_ref.at[c].at[pl.ds(base, ROWS_PER_TILE)])
        if cc == 0:
            plsc.subcore_barrier()


def _aggregate(h_chunks, ei3):
    k = pl.kernel(
        _agg_body,
        out_type=jax.ShapeDtypeStruct((NCHUNK, SP_ROWS, LCHUNK), jnp.float32),
        mesh=plsc.VectorSubcoreMesh(core_axis_name="c", subcore_axis_name="s"),
        scratch_types=[
            pltpu.VMEM_SHARED((SP_ROWS, LCHUNK), jnp.float32),
            pltpu.VMEM((32, 128), jnp.float32),
            pltpu.VMEM((EB, LCHUNK), jnp.float32),
            pltpu.VMEM((EB, LCHUNK), jnp.float32),
            pltpu.VMEM((HB, EB), jnp.int32),
            pltpu.VMEM((HB, EB), jnp.int32),
            pltpu.SemaphoreType.DMA,
            pltpu.SemaphoreType.DMA,
        ],
    )
    return k(h_chunks, ei3)


def _self_body(h_ref, ws_ref, b_ref, o_ref):
    acc = jnp.zeros((BN, D_HID), jnp.float32) + b_ref[...]
    for c in range(NCHUNK):
        acc += jnp.dot(h_ref[c], ws_ref[c], preferred_element_type=jnp.float32)
    o_ref[...] = acc


def _self_term(h_chunks, W_self, b_gnn):
    return pl.pallas_call(
        _self_body,
        grid=(NB,),
        in_specs=[
            pl.BlockSpec((NCHUNK, BN, LCHUNK), lambda i: (0, i, 0)),
            pl.BlockSpec((NCHUNK, LCHUNK, D_HID), lambda i: (0, 0, 0)),
            pl.BlockSpec((1, D_HID), lambda i: (0, 0)),
        ],
        out_specs=pl.BlockSpec((BN, D_HID), lambda i: (i, 0)),
        out_shape=jax.ShapeDtypeStruct((N_NODES, D_HID), jnp.float32),
    )(h_chunks, W_self.reshape(NCHUNK, LCHUNK, D_HID), b_gnn.reshape(1, D_HID))


def _gnn_pool_body(agg_ref, s_ref, wm_ref, batch_ref, out_ref, cnt_ref):
    i = pl.program_id(0)
    acc = s_ref[...]
    for c in range(NCHUNK):
        a = agg_ref[c] if c % 2 == 0 else agg_ref[c] - agg_ref[c - 1]
        acc += jnp.dot(a, wm_ref[c], preferred_element_type=jnp.float32)
    nf = jnp.maximum(acc, 0.0)

    bvec = batch_ref[0, 0, :]
    gids = lax.broadcasted_iota(jnp.int32, (N_GRAPHS, BN), 0)
    maskT = (gids == bvec[None, :]).astype(jnp.float32)
    contrib = jnp.dot(maskT, nf, preferred_element_type=jnp.float32)
    cnt = jnp.sum(maskT, axis=1)

    @pl.when(i == 0)
    def _init():
        out_ref[...] = jnp.zeros_like(out_ref)
        cnt_ref[...] = jnp.zeros_like(cnt_ref)

    out_ref[...] += contrib
    cnt_ref[...] += jnp.broadcast_to(cnt[:, None], (N_GRAPHS, 128))

    @pl.when(i == NB - 1)
    def _fin():
        out_ref[...] = out_ref[...] / jnp.maximum(cnt_ref[:, 0:1], 1.0)


def _gnn_pool(agg_chunks, s, W_msg, batch3):
    return pl.pallas_call(
        _gnn_pool_body,
        grid=(NB,),
        in_specs=[
            pl.BlockSpec((NCHUNK, BN, LCHUNK), lambda i: (0, i, 0)),
            pl.BlockSpec((BN, D_HID), lambda i: (i, 0)),
            pl.BlockSpec((NCHUNK, LCHUNK, D_HID), lambda i: (0, 0, 0)),
            pl.BlockSpec((1, 1, BN), lambda i: (i, 0, 0)),
        ],
        out_specs=pl.BlockSpec((N_GRAPHS, D_HID), lambda i: (0, 0)),
        out_shape=jax.ShapeDtypeStruct((N_GRAPHS, D_HID), jnp.float32),
        scratch_shapes=[pltpu.VMEM((N_GRAPHS, 128), jnp.float32)],
    )(agg_chunks, s, W_msg.reshape(NCHUNK, LCHUNK, D_HID), batch3)


@jax.jit
def kernel(x, edge_index, batch, W_enc, b_enc, W_msg, W_self, b_gnn):
    ei3 = edge_index.astype(jnp.int32).reshape(2, EROWS, EB)
    batch3 = batch.astype(jnp.int32).reshape(NB, 1, BN)

    h_chunks = _encode(x, W_enc, b_enc)
    agg_chunks = _aggregate(h_chunks, ei3)
    s = _self_term(h_chunks, W_self, b_gnn)
    return _gnn_pool(agg_chunks, s, W_msg, batch3)
